# Initial kernel scaffold; baseline (speedup 1.0000x reference)
import jax
import jax.numpy as jnp
from jax.experimental import pallas as pl


def kernel(pos, pin_caps, pin2node_map, branch_u, branch_v, net_branch_start, driver_pin_indices):
    raise NotImplementedError("write your pallas kernel here")


# speedup vs baseline: 58.5591x; 58.5591x over previous
import functools

import jax
import jax.numpy as jnp
from jax import lax
from jax.experimental import pallas as pl
from jax.experimental.pallas import tpu as pltpu
from jax.experimental.pallas import tpu_sc as plsc

_NUM_NODES = 100000
_NUM_PINS = 400000
_NUM_NETS = 50000
_NUM_EDGES = 400000
_R_UNIT = 0.8
_C_UNIT = 0.2
_IGNORE = 100

_NC = 2
_NS = 16
_NW = _NC * _NS
_BLK = 800
_NBLK = _NUM_EDGES // _BLK
_LANES = 16
_VPB = _BLK // _LANES
_NBS_PAD = _NUM_NETS + 8
_BS_ITERS = 16


def _rc_body(posx_hbm, posy_hbm, caps_hbm, p2n_hbm, bu_hbm, bv_hbm, nbs_hbm,
             out_hbm,
             nbs_v, bu_v, bv_v, nu_v, nv_v, xu_v, yu_v, xv_v, yv_v, cv_v,
             out_v, sem):
    wid = lax.axis_index("s") * _NC + lax.axis_index("c")
    pltpu.sync_copy(nbs_hbm, nbs_v)
    iota = lax.iota(jnp.int32, _LANES)

    nblk_mine = (_NBLK - wid + _NW - 1) // _NW

    def block_body(k, carry):
        b = wid + k * _NW
        base = b * _BLK
        pltpu.sync_copy(bu_hbm.at[pl.ds(base, _BLK)], bu_v)
        pltpu.sync_copy(bv_hbm.at[pl.ds(base, _BLK)], bv_v)
        c1 = pltpu.async_copy(p2n_hbm.at[bu_v], nu_v, sem)
        c2 = pltpu.async_copy(p2n_hbm.at[bv_v], nv_v, sem)
        c3 = pltpu.async_copy(caps_hbm.at[bv_v], cv_v, sem)
        c1.wait()
        c2.wait()
        c3.wait()
        c4 = pltpu.async_copy(posx_hbm.at[nu_v], xu_v, sem)
        c5 = pltpu.async_copy(posy_hbm.at[nu_v], yu_v, sem)
        c6 = pltpu.async_copy(posx_hbm.at[nv_v], xv_v, sem)
        c7 = pltpu.async_copy(posy_hbm.at[nv_v], yv_v, sem)
        c4.wait()
        c5.wait()
        c6.wait()
        c7.wait()

        def vec_body(j, vcarry):
            off = j * _LANES
            iota = lax.iota(jnp.int32, _LANES)
            eid = base + off + iota
            xu = xu_v[pl.ds(off, _LANES)]
            yu = yu_v[pl.ds(off, _LANES)]
            xv = xv_v[pl.ds(off, _LANES)]
            yv = yv_v[pl.ds(off, _LANES)]
            cv = cv_v[pl.ds(off, _LANES)]
            wl = jnp.abs(xu - xv) + jnp.abs(yu - yv)

            def bs_step(i, c):
                lo, hi = c
                mid = (lo + hi) // 2
                m = plsc.load_gather(nbs_v, [mid])
                p = m <= eid
                return (jnp.where(p, mid, lo), jnp.where(p, hi, mid))

            lo0 = jnp.zeros((_LANES,), jnp.int32)
            hi0 = jnp.full((_LANES,), _NUM_NETS, jnp.int32)
            lo, hi = lax.fori_loop(0, _BS_ITERS, bs_step, (lo0, hi0))
            s0 = plsc.load_gather(nbs_v, [lo])
            s1 = plsc.load_gather(nbs_v, [lo + 1])
            deg = s1 - s0 + 1
            keep = jnp.where(deg <= _IGNORE, jnp.float32(1.0),
                             jnp.float32(0.0))
            res = (_R_UNIT * wl) * keep
            cap = (_C_UNIT * wl + cv) * keep
            li = off + iota
            plsc.store_scatter(out_v, [2 * li], res)
            plsc.store_scatter(out_v, [2 * li + 1], cap)
            return vcarry

        lax.fori_loop(0, _VPB, vec_body, 0)
        pltpu.sync_copy(out_v, out_hbm.at[pl.ds(2 * base, 2 * _BLK)])
        return carry

    lax.fori_loop(0, nblk_mine, block_body, 0)


@functools.lru_cache(maxsize=1)
def _build():
    mesh = plsc.VectorSubcoreMesh(core_axis_name="c", subcore_axis_name="s")
    return pl.kernel(
        _rc_body,
        out_type=jax.ShapeDtypeStruct((2 * _NUM_EDGES,), jnp.float32),
        mesh=mesh,
        compiler_params=pltpu.CompilerParams(needs_layout_passes=False),
        scratch_types=[
            pltpu.VMEM((_NBS_PAD,), jnp.int32),
            pltpu.VMEM((_BLK,), jnp.int32),
            pltpu.VMEM((_BLK,), jnp.int32),
            pltpu.VMEM((_BLK,), jnp.int32),
            pltpu.VMEM((_BLK,), jnp.int32),
            pltpu.VMEM((_BLK,), jnp.float32),
            pltpu.VMEM((_BLK,), jnp.float32),
            pltpu.VMEM((_BLK,), jnp.float32),
            pltpu.VMEM((_BLK,), jnp.float32),
            pltpu.VMEM((_BLK,), jnp.float32),
            pltpu.VMEM((2 * _BLK,), jnp.float32),
            pltpu.SemaphoreType.DMA,
        ],
    )


def kernel(pos, pin_caps, pin2node_map, branch_u, branch_v, net_branch_start,
           driver_pin_indices):
    posx = pos[:, 0]
    posy = pos[:, 1]
    nbs = jnp.concatenate(
        [net_branch_start,
         jnp.full((_NBS_PAD - _NUM_NETS - 1,), _NUM_EDGES, jnp.int32)])
    out = _build()(posx, posy, pin_caps, pin2node_map, branch_u, branch_v,
                   nbs)
    return out.reshape(_NUM_EDGES, 2)
